# baseline (speedup 1.0000x reference)
import functools

import jax
import jax.numpy as jnp
from jax import lax
from jax.experimental import pallas as pl
from jax.experimental.pallas import tpu as pltpu
from jax.experimental.pallas import tpu_sc as plsc

_B = 4096
_L = 50
_HIDDEN = 128
_CODE = 128
_R = _B * _L

_S = 4
_RS = _R // _S

_NC = 2
_NS = 16
_NW = _NC * _NS
_R_PER_W = _RS // _NW
_CHUNK = 320
_NCHUNK = _R_PER_W // _CHUNK


def _sc_gather(src_idx, emb_table):
    mesh = plsc.VectorSubcoreMesh(core_axis_name="c", subcore_axis_name="s")

    @functools.partial(
        pl.kernel,
        mesh=mesh,
        out_type=jax.ShapeDtypeStruct((_RS, _HIDDEN), jnp.float32),
        scratch_types=[
            pltpu.VMEM((_R_PER_W,), jnp.int32),
            pltpu.VMEM((_CHUNK, _HIDDEN), jnp.float32),
            pltpu.VMEM((_CHUNK, _HIDDEN), jnp.float32),
            pltpu.VMEM((_CHUNK, _HIDDEN), jnp.float32),
            pltpu.SemaphoreType.DMA,
            pltpu.SemaphoreType.DMA,
            pltpu.SemaphoreType.DMA,
            pltpu.SemaphoreType.DMA,
            pltpu.SemaphoreType.DMA,
            pltpu.SemaphoreType.DMA,
        ],
    )
    def gather_kernel(idx_hbm, table_hbm, out_hbm, idx_v,
                      r0, r1, r2, g0, g1, g2, s0, s1, s2):
        wid = lax.axis_index("s") * _NC + lax.axis_index("c")
        base = wid * _R_PER_W
        pltpu.sync_copy(idx_hbm.at[pl.ds(base, _R_PER_W)], idx_v)

        bufs = (r0, r1, r2)
        gsem = (g0, g1, g2)
        ssem = (s0, s1, s2)
        n = _NCHUNK

        def g_copy(c):
            return pltpu.make_async_copy(
                table_hbm.at[idx_v.at[pl.ds(c * _CHUNK, _CHUNK)]],
                bufs[c % 3],
                gsem[c % 3],
            )

        def s_copy(c):
            return pltpu.make_async_copy(
                bufs[c % 3],
                out_hbm.at[pl.ds(base + c * _CHUNK, _CHUNK)],
                ssem[c % 3],
            )

        for c in range(min(3, n)):
            g_copy(c).start()
        for c in range(n):
            g_copy(c).wait()
            s_copy(c).start()
            if c >= 1 and c + 2 < n:
                s_copy(c - 1).wait()
                g_copy(c + 2).start()
        for c in range(max(0, n - 3), n):
            s_copy(c).wait()

    return gather_kernel(src_idx, emb_table)


_BLK = 2048


def _mlp_body(x_ref, w1_ref, b1_ref, w2_ref, b2_ref, o_ref):
    x = x_ref[...].astype(jnp.bfloat16)
    w1 = w1_ref[...].astype(jnp.bfloat16)
    w2 = w2_ref[...].astype(jnp.bfloat16)
    h = jnp.dot(x, w1, preferred_element_type=jnp.float32)
    h = jnp.maximum(h + b1_ref[...], 0.0).astype(jnp.bfloat16)
    o = jnp.dot(h, w2, preferred_element_type=jnp.float32)
    o_ref[...] = jnp.maximum(o + b2_ref[...], 0.0)


def _mlp_body_upd(buf_ref, x_ref, w1_ref, b1_ref, w2_ref, b2_ref, o_ref):
    del buf_ref
    _mlp_body(x_ref, w1_ref, b1_ref, w2_ref, b2_ref, o_ref)


def _tc_mlp_slice(s, buf, enc, W1, b1, W2, b2):
    nblk = _RS // _BLK
    w_specs = [
        pl.BlockSpec((_HIDDEN, _HIDDEN), lambda i: (0, 0)),
        pl.BlockSpec((1, _HIDDEN), lambda i: (0, 0)),
        pl.BlockSpec((_HIDDEN, _CODE), lambda i: (0, 0)),
        pl.BlockSpec((1, _CODE), lambda i: (0, 0)),
    ]
    x_spec = pl.BlockSpec((_BLK, _HIDDEN), lambda i: (i, 0))
    out_spec = pl.BlockSpec((_BLK, _CODE), lambda i, s=s: (i + s * nblk, 0))
    out_shape = jax.ShapeDtypeStruct((_R, _CODE), jnp.float32)
    args = (enc, W1, b1[None, :], W2, b2[None, :])
    if s == 0:
        return pl.pallas_call(
            _mlp_body,
            grid=(nblk,),
            in_specs=[x_spec] + w_specs,
            out_specs=out_spec,
            out_shape=out_shape,
        )(*args)
    return pl.pallas_call(
        _mlp_body_upd,
        grid=(nblk,),
        in_specs=[pl.BlockSpec(memory_space=pl.ANY), x_spec] + w_specs,
        out_specs=out_spec,
        out_shape=out_shape,
        input_output_aliases={0: 0},
    )(buf, *args)


def kernel(src_seq, emb_table, W1, b1, W2, b2):
    idx = src_seq.T.reshape(_R).astype(jnp.int32)
    encs = [
        _sc_gather(lax.slice(idx, (s * _RS,), ((s + 1) * _RS,)), emb_table)
        for s in range(_S)
    ]
    out = None
    for s in range(_S):
        out = _tc_mlp_slice(s, out, encs[s], W1, b1, W2, b2)
    return out.reshape(_L, _B, _CODE).transpose(1, 0, 2)


# speedup vs baseline: 1.0164x; 1.0164x over previous
import functools

import jax
import jax.numpy as jnp
from jax import lax
from jax.experimental import pallas as pl
from jax.experimental.pallas import tpu as pltpu
from jax.experimental.pallas import tpu_sc as plsc

_B = 4096
_L = 50
_HIDDEN = 128
_CODE = 128
_R = _B * _L

_S = 5
_RS = _R // _S

_NC = 2
_NS = 16
_NW = _NC * _NS
_R_PER_W = _RS // _NW
_CHUNK = 320
_NCHUNK = _R_PER_W // _CHUNK


def _sc_gather(src_idx, emb_table):
    mesh = plsc.VectorSubcoreMesh(core_axis_name="c", subcore_axis_name="s")

    @functools.partial(
        pl.kernel,
        mesh=mesh,
        out_type=jax.ShapeDtypeStruct((_RS, _HIDDEN), jnp.float32),
        scratch_types=[
            pltpu.VMEM((_R_PER_W,), jnp.int32),
            pltpu.VMEM((_CHUNK, _HIDDEN), jnp.float32),
            pltpu.VMEM((_CHUNK, _HIDDEN), jnp.float32),
            pltpu.VMEM((_CHUNK, _HIDDEN), jnp.float32),
            pltpu.SemaphoreType.DMA,
            pltpu.SemaphoreType.DMA,
            pltpu.SemaphoreType.DMA,
            pltpu.SemaphoreType.DMA,
            pltpu.SemaphoreType.DMA,
            pltpu.SemaphoreType.DMA,
        ],
    )
    def gather_kernel(idx_hbm, table_hbm, out_hbm, idx_v,
                      r0, r1, r2, g0, g1, g2, s0, s1, s2):
        wid = lax.axis_index("s") * _NC + lax.axis_index("c")
        base = wid * _R_PER_W
        pltpu.sync_copy(idx_hbm.at[pl.ds(base, _R_PER_W)], idx_v)

        bufs = (r0, r1, r2)
        gsem = (g0, g1, g2)
        ssem = (s0, s1, s2)
        n = _NCHUNK

        def g_copy(c):
            return pltpu.make_async_copy(
                table_hbm.at[idx_v.at[pl.ds(c * _CHUNK, _CHUNK)]],
                bufs[c % 3],
                gsem[c % 3],
            )

        def s_copy(c):
            return pltpu.make_async_copy(
                bufs[c % 3],
                out_hbm.at[pl.ds(base + c * _CHUNK, _CHUNK)],
                ssem[c % 3],
            )

        for c in range(min(3, n)):
            g_copy(c).start()
        for c in range(n):
            g_copy(c).wait()
            s_copy(c).start()
            if c >= 1 and c + 2 < n:
                s_copy(c - 1).wait()
                g_copy(c + 2).start()
        for c in range(max(0, n - 3), n):
            s_copy(c).wait()

    return gather_kernel(src_idx, emb_table)


_BLK = 2048


def _mlp_body(x_ref, w1_ref, b1_ref, w2_ref, b2_ref, o_ref):
    x = x_ref[...].astype(jnp.bfloat16)
    w1 = w1_ref[...].astype(jnp.bfloat16)
    w2 = w2_ref[...].astype(jnp.bfloat16)
    h = jnp.dot(x, w1, preferred_element_type=jnp.float32)
    h = jnp.maximum(h + b1_ref[...], 0.0).astype(jnp.bfloat16)
    o = jnp.dot(h, w2, preferred_element_type=jnp.float32)
    o_ref[...] = jnp.maximum(o + b2_ref[...], 0.0)


def _mlp_body_upd(buf_ref, x_ref, w1_ref, b1_ref, w2_ref, b2_ref, o_ref):
    del buf_ref
    _mlp_body(x_ref, w1_ref, b1_ref, w2_ref, b2_ref, o_ref)


def _tc_mlp_slice(s, buf, enc, W1, b1, W2, b2):
    nblk = _RS // _BLK
    w_specs = [
        pl.BlockSpec((_HIDDEN, _HIDDEN), lambda i: (0, 0)),
        pl.BlockSpec((1, _HIDDEN), lambda i: (0, 0)),
        pl.BlockSpec((_HIDDEN, _CODE), lambda i: (0, 0)),
        pl.BlockSpec((1, _CODE), lambda i: (0, 0)),
    ]
    x_spec = pl.BlockSpec((_BLK, _HIDDEN), lambda i: (i, 0))
    out_spec = pl.BlockSpec((_BLK, _CODE), lambda i, s=s: (i + s * nblk, 0))
    out_shape = jax.ShapeDtypeStruct((_R, _CODE), jnp.float32)
    args = (enc, W1, b1[None, :], W2, b2[None, :])
    if s == 0:
        return pl.pallas_call(
            _mlp_body,
            grid=(nblk,),
            in_specs=[x_spec] + w_specs,
            out_specs=out_spec,
            out_shape=out_shape,
        )(*args)
    return pl.pallas_call(
        _mlp_body_upd,
        grid=(nblk,),
        in_specs=[pl.BlockSpec(memory_space=pl.ANY), x_spec] + w_specs,
        out_specs=out_spec,
        out_shape=out_shape,
        input_output_aliases={0: 0},
    )(buf, *args)


def kernel(src_seq, emb_table, W1, b1, W2, b2):
    idx = src_seq.T.reshape(_R).astype(jnp.int32)
    encs = [
        _sc_gather(lax.slice(idx, (s * _RS,), ((s + 1) * _RS,)), emb_table)
        for s in range(_S)
    ]
    out = None
    for s in range(_S):
        out = _tc_mlp_slice(s, out, encs[s], W1, b1, W2, b2)
    return out.reshape(_L, _B, _CODE).transpose(1, 0, 2)
